# baseline (speedup 1.0000x reference)
import functools

import jax
import jax.numpy as jnp
from jax import lax
from jax.experimental import pallas as pl
from jax.experimental.pallas import tpu as pltpu
from jax.experimental.pallas import tpu_sc as plsc

_NC, _NS = 2, 16
_NW = _NC * _NS
_GB = 16
_KC = 3
_NBUF = 2


def _sc_scatter_body(ids_hbm, zeros_hbm, out_hbm, ids_v, buf0, buf1, sem0, sem1):
    wid = lax.axis_index("s") * _NC + lax.axis_index("c")
    tpw = ids_v.shape[0] // (_KC * 16)
    n = zeros_hbm.shape[1]
    base = wid * tpw
    ng = tpw // _GB
    pltpu.sync_copy(ids_hbm.at[pl.ds(base * (_KC * 16), tpw * (_KC * 16))], ids_v)
    pltpu.sync_copy(zeros_hbm, buf0)
    pltpu.sync_copy(zeros_hbm, buf1)
    zeros16 = jnp.zeros((16,), jnp.float32)
    ones16 = jnp.ones((16,), jnp.float32)
    bufs = (buf0, buf1)
    sems = (sem0, sem1)

    def _scatter_group(buf, g, val):
        for r in range(_GB):
            lt = g * _GB + r
            row = jnp.full((16,), r, jnp.int32)
            for k in range(_KC):
                idx = ids_v[pl.ds(lt * (_KC * 16) + k * 16, 16)]
                plsc.store_scatter(buf, [row, idx], val)

    def body(i, carry):
        for b in range(_NBUF):
            buf, sem = bufs[b], sems[b]
            g = i * _NBUF + b

            @pl.when(i > 0)
            def _():
                pltpu.make_async_copy(
                    buf, out_hbm.at[pl.ds(0, _GB)], sem).wait()
                _scatter_group(buf, g - _NBUF, zeros16)

            _scatter_group(buf, g, ones16)
            pltpu.async_copy(
                buf, out_hbm.at[pl.ds(base + g * _GB, _GB)], sem)
        return carry

    lax.fori_loop(0, ng // _NBUF, body, 0)
    for b in range(_NBUF):
        pltpu.make_async_copy(
            bufs[b], out_hbm.at[pl.ds(0, _GB)], sems[b]).wait()


def _build_onehot_sc(ids_pad, t, n):
    mesh = plsc.VectorSubcoreMesh(
        core_axis_name="c", subcore_axis_name="s",
        num_cores=_NC, num_subcores=_NS)
    tpw = t // _NW
    zeros = jnp.zeros((_GB, n), jnp.float32)
    sc_kernel = functools.partial(
        pl.kernel,
        out_type=jax.ShapeDtypeStruct((t, n), jnp.float32),
        mesh=mesh,
        scratch_types=[
            pltpu.VMEM((tpw * _KC * 16,), jnp.int32),
            pltpu.VMEM((_GB, n), jnp.float32),
            pltpu.VMEM((_GB, n), jnp.float32),
            pltpu.SemaphoreType.DMA,
            pltpu.SemaphoreType.DMA,
        ],
        compiler_params=pltpu.CompilerParams(needs_layout_passes=False),
    )(_sc_scatter_body)
    return sc_kernel(ids_pad, zeros)


def _proj_ln_body(oh_ref, wt_ref, g_ref, b_ref, o_ref):
    oh = oh_ref[...].astype(wt_ref.dtype)
    x = jnp.dot(oh, wt_ref[...], preferred_element_type=jnp.float32)
    mean = jnp.mean(x, axis=1, keepdims=True)
    var = jnp.mean(x * x, axis=1, keepdims=True) - mean * mean
    inv = lax.rsqrt(var + 1e-5)
    o_ref[...] = (x - mean) * inv * g_ref[...] + b_ref[...]


def _proj_ln_body_alias(prev_ref, oh_ref, wt_ref, g_ref, b_ref, o_ref):
    del prev_ref
    _proj_ln_body(oh_ref, wt_ref, g_ref, b_ref, o_ref)


def _proj_ln_chunk(onehot, wt, gamma, beta, t, n, h, tb, blk_off, prev):
    grid = onehot.shape[0] // tb
    in_specs = [
        pl.BlockSpec((tb, n), lambda i: (i, 0)),
        pl.BlockSpec((n, h), lambda i: (0, 0)),
        pl.BlockSpec((1, h), lambda i: (0, 0)),
        pl.BlockSpec((1, h), lambda i: (0, 0)),
    ]
    args = (onehot, wt, gamma, beta)
    body = _proj_ln_body
    aliases = {}
    if prev is not None:
        in_specs = [pl.BlockSpec(memory_space=pl.ANY)] + in_specs
        args = (prev,) + args
        body = _proj_ln_body_alias
        aliases = {0: 0}
    return pl.pallas_call(
        body,
        grid=(grid,),
        in_specs=in_specs,
        out_specs=pl.BlockSpec((tb, h), lambda i, o=blk_off: (i + o, 0)),
        out_shape=jax.ShapeDtypeStruct((t, h), jnp.float32),
        input_output_aliases=aliases,
    )(*args)


def kernel(input_ids, proj_w, ln_gamma, ln_beta):
    b, s, w = input_ids.shape
    h, n = proj_w.shape
    t = b * s
    ids = input_ids.reshape(t, w).astype(jnp.int32)
    pad = (-w) % 16
    wp = w + pad
    ids_pad = jnp.concatenate(
        [ids, jnp.broadcast_to(ids[:, :1], (t, pad))], axis=1
    ).reshape(t * wp)
    wt = proj_w.T.astype(jnp.bfloat16)
    gamma = ln_gamma.reshape(1, h)
    beta = ln_beta.reshape(1, h)
    tb = 512
    while t % tb:
        tb //= 2
    grain = _NW * _GB
    if t % (16 * grain) == 0 and (t // 16) % tb == 0:
        sizes = [(5 * t) // 16, (11 * t) // 16]
    elif t % (4 * grain) == 0 and (t // 4) % tb == 0:
        sizes = [t // 4, (3 * t) // 4]
    else:
        sizes = [t]
    onehots = []
    off = 0
    for tc in sizes:
        ids_c = lax.dynamic_slice(ids_pad, (off * wp,), (tc * wp,))
        onehots.append(_build_onehot_sc(ids_c, tc, n))
        off += tc
    out = None
    off = 0
    for tc, oh in zip(sizes, onehots):
        out = _proj_ln_chunk(
            oh, wt, gamma, beta, t, n, h, tb, off // tb, out)
        off += tc
    return out.reshape(b, s, h)


# speedup vs baseline: 1.0083x; 1.0083x over previous
import functools

import jax
import jax.numpy as jnp
from jax import lax
from jax.experimental import pallas as pl
from jax.experimental.pallas import tpu as pltpu
from jax.experimental.pallas import tpu_sc as plsc

_NC, _NS = 2, 16
_NW = _NC * _NS
_GB = 16
_KC = 3
_NBUF = 2


def _sc_scatter_body(ids_hbm, zeros_hbm, out_hbm, ids_v, buf0, buf1, sem0, sem1):
    wid = lax.axis_index("s") * _NC + lax.axis_index("c")
    tpw = ids_v.shape[0] // (_KC * 16)
    n = zeros_hbm.shape[1]
    base = wid * tpw
    ng = tpw // _GB
    pltpu.sync_copy(ids_hbm.at[pl.ds(base * (_KC * 16), tpw * (_KC * 16))], ids_v)
    pltpu.sync_copy(zeros_hbm, buf0)
    pltpu.sync_copy(zeros_hbm, buf1)
    zeros16 = jnp.zeros((16,), jnp.float32)
    ones16 = jnp.ones((16,), jnp.float32)
    bufs = (buf0, buf1)
    sems = (sem0, sem1)

    def _scatter_group(buf, g, val):
        for r in range(_GB):
            lt = g * _GB + r
            row = jnp.full((16,), r, jnp.int32)
            for k in range(_KC):
                idx = ids_v[pl.ds(lt * (_KC * 16) + k * 16, 16)]
                plsc.store_scatter(buf, [row, idx], val)

    def body(i, carry):
        for b in range(_NBUF):
            buf, sem = bufs[b], sems[b]
            g = i * _NBUF + b

            @pl.when(i > 0)
            def _():
                pltpu.make_async_copy(
                    buf, out_hbm.at[pl.ds(0, _GB)], sem).wait()
                _scatter_group(buf, g - _NBUF, zeros16)

            _scatter_group(buf, g, ones16)
            pltpu.async_copy(
                buf, out_hbm.at[pl.ds(base + g * _GB, _GB)], sem)
        return carry

    lax.fori_loop(0, ng // _NBUF, body, 0)
    for b in range(_NBUF):
        pltpu.make_async_copy(
            bufs[b], out_hbm.at[pl.ds(0, _GB)], sems[b]).wait()


def _build_onehot_sc(ids_pad, t, n):
    mesh = plsc.VectorSubcoreMesh(
        core_axis_name="c", subcore_axis_name="s",
        num_cores=_NC, num_subcores=_NS)
    tpw = t // _NW
    zeros = jnp.zeros((_GB, n), jnp.float32)
    sc_kernel = functools.partial(
        pl.kernel,
        out_type=jax.ShapeDtypeStruct((t, n), jnp.float32),
        mesh=mesh,
        scratch_types=[
            pltpu.VMEM((tpw * _KC * 16,), jnp.int32),
            pltpu.VMEM((_GB, n), jnp.float32),
            pltpu.VMEM((_GB, n), jnp.float32),
            pltpu.SemaphoreType.DMA,
            pltpu.SemaphoreType.DMA,
        ],
        compiler_params=pltpu.CompilerParams(needs_layout_passes=False),
    )(_sc_scatter_body)
    return sc_kernel(ids_pad, zeros)


def _proj_ln_body(oh_ref, wt_ref, g_ref, b_ref, o_ref):
    oh = oh_ref[...].astype(wt_ref.dtype)
    x = jnp.dot(oh, wt_ref[...], preferred_element_type=jnp.float32)
    mean = jnp.mean(x, axis=1, keepdims=True)
    var = jnp.mean(x * x, axis=1, keepdims=True) - mean * mean
    inv = lax.rsqrt(var + 1e-5)
    o_ref[...] = (x - mean) * inv * g_ref[...] + b_ref[...]


def _proj_ln_body_alias(prev_ref, oh_ref, wt_ref, g_ref, b_ref, o_ref):
    del prev_ref
    _proj_ln_body(oh_ref, wt_ref, g_ref, b_ref, o_ref)


def _proj_ln_chunk(onehot, wt, gamma, beta, t, n, h, tb, blk_off, prev):
    grid = onehot.shape[0] // tb
    in_specs = [
        pl.BlockSpec((tb, n), lambda i: (i, 0)),
        pl.BlockSpec((n, h), lambda i: (0, 0)),
        pl.BlockSpec((1, h), lambda i: (0, 0)),
        pl.BlockSpec((1, h), lambda i: (0, 0)),
    ]
    args = (onehot, wt, gamma, beta)
    body = _proj_ln_body
    aliases = {}
    if prev is not None:
        in_specs = [pl.BlockSpec(memory_space=pl.ANY)] + in_specs
        args = (prev,) + args
        body = _proj_ln_body_alias
        aliases = {0: 0}
    return pl.pallas_call(
        body,
        grid=(grid,),
        in_specs=in_specs,
        out_specs=pl.BlockSpec((tb, h), lambda i, o=blk_off: (i + o, 0)),
        out_shape=jax.ShapeDtypeStruct((t, h), jnp.float32),
        input_output_aliases=aliases,
    )(*args)


def kernel(input_ids, proj_w, ln_gamma, ln_beta):
    b, s, w = input_ids.shape
    h, n = proj_w.shape
    t = b * s
    ids = input_ids.reshape(t, w).astype(jnp.int32)
    pad = (-w) % 16
    wp = w + pad
    ids_pad = jnp.concatenate(
        [ids, jnp.broadcast_to(ids[:, :1], (t, pad))], axis=1
    ).reshape(t * wp)
    wt = proj_w.T.astype(jnp.bfloat16)
    gamma = ln_gamma.reshape(1, h)
    beta = ln_beta.reshape(1, h)
    tb = 512
    while t % tb:
        tb //= 2
    grain = _NW * _GB
    if t % (4 * grain) == 0 and (t // 4) % tb == 0:
        sizes = [t // 4, (3 * t) // 4]
    else:
        sizes = [t]
    onehots = []
    off = 0
    for tc in sizes:
        ids_c = lax.dynamic_slice(ids_pad, (off * wp,), (tc * wp,))
        onehots.append(_build_onehot_sc(ids_c, tc, n))
        off += tc
    out = None
    off = 0
    for tc, oh in zip(sizes, onehots):
        out = _proj_ln_chunk(
            oh, wt, gamma, beta, t, n, h, tb, off // tb, out)
        off += tc
    return out.reshape(b, s, h)
